# baseline (speedup 1.0000x reference)
import jax
import jax.numpy as jnp
from jax import lax
from jax.experimental import pallas as pl
from jax.experimental.pallas import tpu as pltpu
from jax.experimental.pallas import tpu_sc as plsc

B, N, T = 8, 5000, 50
LANES = 16
N_PAD = 5120
CHUNK = N_PAD // 4
T_PAD = 80
N_GROUPS = CHUNK // LANES
NEG_INF = float("-inf")
BIG = 2**30


def _sc_body(ps_s_hbm, ps_e_hbm, ts_s_hbm, ts_e_hbm,
             out_idx_hbm, out_val_hbm,
             ps_s_v, ps_e_v, ts_s_v, ts_e_v, oidx_v, oval_v,
             stage_v, mrg_v, sh_win, sem):
    c = lax.axis_index("c")
    s = lax.axis_index("s")
    bloc = s // 4
    chunk = s % 4
    b = c * 4 + bloc
    w = b * 4 + chunk
    base_n = chunk * CHUNK

    cp1 = pltpu.make_async_copy(ps_s_hbm.at[w], ps_s_v, sem)
    cp1.start()
    cp2 = pltpu.make_async_copy(ps_e_hbm.at[w], ps_e_v.at[pl.ds(0, CHUNK)], sem)
    cp2.start()
    cp3 = pltpu.make_async_copy(ts_s_hbm.at[b], ts_s_v, sem)
    cp3.start()
    cp4 = pltpu.make_async_copy(ts_e_hbm.at[b], ts_e_v, sem)
    cp4.start()
    cp1.wait()
    cp2.wait()
    cp3.wait()
    cp4.wait()

    lane_iota = lax.broadcasted_iota(jnp.int32, (LANES,), 0)
    minus_inf = jnp.float32(NEG_INF)

    def init_step(g, _):
        idxv = lane_iota + (base_n + g * LANES)
        pe = ps_e_v[pl.ds(g * LANES, LANES)]
        ps_e_v[pl.ds(g * LANES, LANES)] = jnp.where(
            idxv < N, pe, minus_inf)
        return 0

    lax.fori_loop(0, N_GROUPS, init_step, 0, unroll=8)

    def t_step(t, _):
        ts = ts_s_v[pl.ds(t, LANES)][0]
        te = ts_e_v[pl.ds(t, LANES)][0]
        lt = te - ts

        def g_step(g, carry):
            bv, bg = carry
            base = g * LANES
            ps = ps_s_v[pl.ds(base, LANES)]
            pe = ps_e_v[pl.ds(base, LANES)]
            inter = jnp.maximum(
                jnp.minimum(te, pe) - jnp.maximum(ts, ps), 0.0)
            lp = pe - ps
            union = lt + lp - inter
            iou = inter / jnp.maximum(union, 1e-8)
            enclose = jnp.maximum(te, pe) - jnp.minimum(ts, ps)
            score = iou - (enclose - union) / jnp.maximum(enclose, 1e-8)
            upd = score > bv
            bv = jnp.where(upd, score, bv)
            bg = jnp.where(upd, g, bg)
            return bv, bg

        bv, bg = lax.fori_loop(
            0, N_GROUPS, g_step,
            (jnp.full((LANES,), NEG_INF, jnp.float32),
             jnp.zeros((LANES,), jnp.int32)), unroll=8)
        bi = bg * LANES + lane_iota

        m = bv[0]
        for l in range(1, LANES):
            m = jnp.maximum(m, bv[l])
        lidx = jnp.int32(BIG)
        for l in range(LANES):
            lidx = jnp.where(bv[l] == m, jnp.minimum(lidx, bi[l]), lidx)

        parity = t % 2
        stage_v[pl.ds(0, LANES)] = jnp.full((LANES,), m, jnp.float32)
        stage_v[pl.ds(LANES, LANES)] = jnp.full(
            (LANES,), (lidx + base_n).astype(jnp.float32), jnp.float32)
        slot = parity * (16 * 2 * LANES) + s * (2 * LANES)
        pltpu.sync_copy(stage_v, sh_win.at[pl.ds(slot, 2 * LANES)])
        plsc.subcore_barrier()

        roff = parity * (16 * 2 * LANES) + bloc * (4 * 2 * LANES)
        pltpu.sync_copy(sh_win.at[pl.ds(roff, 4 * 2 * LANES)], mrg_v)
        mvals = [mrg_v[pl.ds(k * 2 * LANES, LANES)][0] for k in range(4)]
        midxs = [mrg_v[pl.ds(k * 2 * LANES + LANES, LANES)][0]
                 .astype(jnp.int32) for k in range(4)]
        mg = mvals[0]
        for k in range(1, 4):
            mg = jnp.maximum(mg, mvals[k])
        gidx = jnp.int32(BIG)
        for k in range(4):
            gidx = jnp.where(mvals[k] == mg,
                             jnp.minimum(gidx, midxs[k]), gidx)

        loc = gidx - base_n

        @pl.when(jnp.logical_and(loc >= 0, loc < CHUNK))
        def _():
            lane0 = lane_iota == 0
            vu = ps_e_v[pl.ds(loc, LANES)]
            ps_e_v[pl.ds(loc, LANES)] = jnp.where(lane0, minus_inf, vu)

        @pl.when(chunk == 0)
        def _():
            lane0 = lane_iota == 0
            vi = oidx_v[pl.ds(t, LANES)]
            oidx_v[pl.ds(t, LANES)] = jnp.where(lane0, gidx, vi)
            vv = oval_v[pl.ds(t, LANES)]
            oval_v[pl.ds(t, LANES)] = jnp.where(lane0, mg, vv)

        return 0

    lax.fori_loop(0, T, t_step, 0)

    @pl.when(chunk == 0)
    def _():
        pltpu.sync_copy(oidx_v, out_idx_hbm.at[b])
        pltpu.sync_copy(oval_v, out_val_hbm.at[b])


@jax.jit
def kernel(pred_logits, pred_segments, tgt_segments, prediction_duration):
    del pred_logits
    scale = prediction_duration[:, None, None]
    ps = pred_segments * scale
    ts = tgt_segments * scale
    ps_s = jnp.pad(ps[..., 0], ((0, 0), (0, N_PAD - N))).reshape(B * 4, CHUNK)
    ps_e = jnp.pad(ps[..., 1], ((0, 0), (0, N_PAD - N))).reshape(B * 4, CHUNK)
    ts_s = jnp.pad(ts[..., 0], ((0, 0), (0, T_PAD - T)))
    ts_e = jnp.pad(ts[..., 1], ((0, 0), (0, T_PAD - T)))

    mesh = plsc.VectorSubcoreMesh(core_axis_name="c", subcore_axis_name="s")
    run = pl.kernel(
        _sc_body,
        out_type=(jax.ShapeDtypeStruct((B, T_PAD), jnp.int32),
                  jax.ShapeDtypeStruct((B, T_PAD), jnp.float32)),
        mesh=mesh,
        scratch_types=[
            pltpu.VMEM((CHUNK,), jnp.float32),
            pltpu.VMEM((CHUNK + LANES,), jnp.float32),
            pltpu.VMEM((T_PAD,), jnp.float32),
            pltpu.VMEM((T_PAD,), jnp.float32),
            pltpu.VMEM((T_PAD,), jnp.int32),
            pltpu.VMEM((T_PAD,), jnp.float32),
            pltpu.VMEM((2 * LANES,), jnp.float32),
            pltpu.VMEM((4 * 2 * LANES,), jnp.float32),
            pltpu.VMEM_SHARED((2 * 16 * 2 * LANES,), jnp.float32),
            pltpu.SemaphoreType.DMA,
        ],
    )
    out_idx, out_val = run(ps_s, ps_e, ts_s, ts_e)
    return (out_idx[:, :T].astype(jnp.int64),
            out_val[:, :T])


# speedup vs baseline: 1.0095x; 1.0095x over previous
import jax
import jax.numpy as jnp
from jax import lax
from jax.experimental import pallas as pl
from jax.experimental.pallas import tpu as pltpu
from jax.experimental.pallas import tpu_sc as plsc

B, N, T = 8, 5000, 50
LANES = 16
N_PAD = 5120
CHUNK = N_PAD // 4
T_PAD = 80
N_GROUPS = CHUNK // LANES
NEG_INF = float("-inf")
BIG = 2**30


def _sc_body(ps_s_hbm, ps_e_hbm, ts_s_hbm, ts_e_hbm,
             out_idx_hbm, out_val_hbm,
             ps_s_v, ps_e_v, ts_s_v, ts_e_v, oidx_v, oval_v,
             stage_v, mrg_v, sh_win, sem):
    c = lax.axis_index("c")
    s = lax.axis_index("s")
    bloc = s // 4
    chunk = s % 4
    b = c * 4 + bloc
    w = b * 4 + chunk
    base_n = chunk * CHUNK

    cp1 = pltpu.make_async_copy(ps_s_hbm.at[w], ps_s_v, sem)
    cp1.start()
    cp2 = pltpu.make_async_copy(ps_e_hbm.at[w], ps_e_v.at[pl.ds(0, CHUNK)], sem)
    cp2.start()
    cp3 = pltpu.make_async_copy(ts_s_hbm.at[b], ts_s_v, sem)
    cp3.start()
    cp4 = pltpu.make_async_copy(ts_e_hbm.at[b], ts_e_v, sem)
    cp4.start()
    cp1.wait()
    cp2.wait()
    cp3.wait()
    cp4.wait()

    lane_iota = lax.broadcasted_iota(jnp.int32, (LANES,), 0)
    minus_inf = jnp.float32(NEG_INF)

    def init_step(g, _):
        idxv = lane_iota + (base_n + g * LANES)
        pe = ps_e_v[pl.ds(g * LANES, LANES)]
        ps_e_v[pl.ds(g * LANES, LANES)] = jnp.where(
            idxv < N, pe, minus_inf)
        return 0

    lax.fori_loop(0, N_GROUPS, init_step, 0, unroll=8)

    def t_step(t, _):
        ts = ts_s_v[pl.ds(t, LANES)][0]
        te = ts_e_v[pl.ds(t, LANES)][0]
        lt = te - ts

        def g_step(g, carry):
            bv, bg = carry
            base = g * LANES
            ps = ps_s_v[pl.ds(base, LANES)]
            pe = ps_e_v[pl.ds(base, LANES)]
            inter = jnp.maximum(
                jnp.minimum(te, pe) - jnp.maximum(ts, ps), 0.0)
            lp = pe - ps
            union = lt + lp - inter
            iou = inter / jnp.maximum(union, 1e-8)
            enclose = jnp.maximum(te, pe) - jnp.minimum(ts, ps)
            score = iou - (enclose - union) / jnp.maximum(enclose, 1e-8)
            upd = score > bv
            bv = jnp.where(upd, score, bv)
            bg = jnp.where(upd, g, bg)
            return bv, bg

        bv, bg = lax.fori_loop(
            0, N_GROUPS, g_step,
            (jnp.full((LANES,), NEG_INF, jnp.float32),
             jnp.zeros((LANES,), jnp.int32)), unroll=8)
        bi = bg * LANES + lane_iota

        bvf = jnp.maximum(bv, lax.rev(bv, (0,)))
        m = bvf[0]
        for l in range(1, LANES // 2):
            m = jnp.maximum(m, bvf[l])
        cand = jnp.where(bv == m, bi, BIG)
        candf = jnp.minimum(cand, lax.rev(cand, (0,)))
        lidx = candf[0]
        for l in range(1, LANES // 2):
            lidx = jnp.minimum(lidx, candf[l])

        parity = t % 2
        stage_v[pl.ds(0, LANES)] = jnp.full((LANES,), m, jnp.float32)
        stage_v[pl.ds(LANES, LANES)] = jnp.full(
            (LANES,), (lidx + base_n).astype(jnp.float32), jnp.float32)
        slot = parity * (16 * 2 * LANES) + s * (2 * LANES)
        pltpu.sync_copy(stage_v, sh_win.at[pl.ds(slot, 2 * LANES)])
        plsc.subcore_barrier()

        roff = parity * (16 * 2 * LANES) + bloc * (4 * 2 * LANES)
        pltpu.sync_copy(sh_win.at[pl.ds(roff, 4 * 2 * LANES)], mrg_v)
        mvals = [mrg_v[pl.ds(k * 2 * LANES, LANES)][0] for k in range(4)]
        midxs = [mrg_v[pl.ds(k * 2 * LANES + LANES, LANES)][0]
                 .astype(jnp.int32) for k in range(4)]
        mg = mvals[0]
        for k in range(1, 4):
            mg = jnp.maximum(mg, mvals[k])
        gidx = jnp.int32(BIG)
        for k in range(4):
            gidx = jnp.where(mvals[k] == mg,
                             jnp.minimum(gidx, midxs[k]), gidx)

        loc = gidx - base_n

        @pl.when(jnp.logical_and(loc >= 0, loc < CHUNK))
        def _():
            lane0 = lane_iota == 0
            vu = ps_e_v[pl.ds(loc, LANES)]
            ps_e_v[pl.ds(loc, LANES)] = jnp.where(lane0, minus_inf, vu)

        @pl.when(chunk == 0)
        def _():
            lane0 = lane_iota == 0
            vi = oidx_v[pl.ds(t, LANES)]
            oidx_v[pl.ds(t, LANES)] = jnp.where(lane0, gidx, vi)
            vv = oval_v[pl.ds(t, LANES)]
            oval_v[pl.ds(t, LANES)] = jnp.where(lane0, mg, vv)

        return 0

    lax.fori_loop(0, T, t_step, 0)

    @pl.when(chunk == 0)
    def _():
        pltpu.sync_copy(oidx_v, out_idx_hbm.at[b])
        pltpu.sync_copy(oval_v, out_val_hbm.at[b])


@jax.jit
def kernel(pred_logits, pred_segments, tgt_segments, prediction_duration):
    del pred_logits
    scale = prediction_duration[:, None, None]
    ps = pred_segments * scale
    ts = tgt_segments * scale
    ps_s = jnp.pad(ps[..., 0], ((0, 0), (0, N_PAD - N))).reshape(B * 4, CHUNK)
    ps_e = jnp.pad(ps[..., 1], ((0, 0), (0, N_PAD - N))).reshape(B * 4, CHUNK)
    ts_s = jnp.pad(ts[..., 0], ((0, 0), (0, T_PAD - T)))
    ts_e = jnp.pad(ts[..., 1], ((0, 0), (0, T_PAD - T)))

    mesh = plsc.VectorSubcoreMesh(core_axis_name="c", subcore_axis_name="s")
    run = pl.kernel(
        _sc_body,
        out_type=(jax.ShapeDtypeStruct((B, T_PAD), jnp.int32),
                  jax.ShapeDtypeStruct((B, T_PAD), jnp.float32)),
        mesh=mesh,
        scratch_types=[
            pltpu.VMEM((CHUNK,), jnp.float32),
            pltpu.VMEM((CHUNK + LANES,), jnp.float32),
            pltpu.VMEM((T_PAD,), jnp.float32),
            pltpu.VMEM((T_PAD,), jnp.float32),
            pltpu.VMEM((T_PAD,), jnp.int32),
            pltpu.VMEM((T_PAD,), jnp.float32),
            pltpu.VMEM((2 * LANES,), jnp.float32),
            pltpu.VMEM((4 * 2 * LANES,), jnp.float32),
            pltpu.VMEM_SHARED((2 * 16 * 2 * LANES,), jnp.float32),
            pltpu.SemaphoreType.DMA,
        ],
    )
    out_idx, out_val = run(ps_s, ps_e, ts_s, ts_e)
    return (out_idx[:, :T].astype(jnp.int64),
            out_val[:, :T])


# speedup vs baseline: 1.0158x; 1.0062x over previous
import jax
import jax.numpy as jnp
from jax import lax
from jax.experimental import pallas as pl
from jax.experimental.pallas import tpu as pltpu
from jax.experimental.pallas import tpu_sc as plsc

B, N, T = 8, 5000, 50
LANES = 16
N_PAD = 5120
CHUNK = N_PAD // 4
T_PAD = 80
N_GROUPS = CHUNK // LANES
NEG_INF = float("-inf")
BIG = 2**30


def _sc_body(ps_s_hbm, ps_e_hbm, ts_s_hbm, ts_e_hbm,
             out_idx_hbm, out_val_hbm,
             ps_s_v, ps_e_v, ts_s_v, ts_e_v, oidx_v, oval_v,
             stage_v, mrg_v, sh_win, sem):
    c = lax.axis_index("c")
    s = lax.axis_index("s")
    bloc = s // 4
    chunk = s % 4
    b = c * 4 + bloc
    w = b * 4 + chunk
    base_n = chunk * CHUNK

    cp1 = pltpu.make_async_copy(ps_s_hbm.at[w], ps_s_v, sem)
    cp1.start()
    cp2 = pltpu.make_async_copy(ps_e_hbm.at[w], ps_e_v.at[pl.ds(0, CHUNK)], sem)
    cp2.start()
    cp3 = pltpu.make_async_copy(ts_s_hbm.at[b], ts_s_v, sem)
    cp3.start()
    cp4 = pltpu.make_async_copy(ts_e_hbm.at[b], ts_e_v, sem)
    cp4.start()
    cp1.wait()
    cp2.wait()
    cp3.wait()
    cp4.wait()

    lane_iota = lax.broadcasted_iota(jnp.int32, (LANES,), 0)
    minus_inf = jnp.float32(NEG_INF)

    def init_step(g, _):
        idxv = lane_iota + (base_n + g * LANES)
        pe = ps_e_v[pl.ds(g * LANES, LANES)]
        ps_e_v[pl.ds(g * LANES, LANES)] = jnp.where(
            idxv < N, pe, minus_inf)
        return 0

    lax.fori_loop(0, N_GROUPS, init_step, 0, unroll=8)

    def t_step(t, _):
        ts = ts_s_v[pl.ds(t, LANES)][0]
        te = ts_e_v[pl.ds(t, LANES)][0]
        lt = te - ts

        def g_step(g, carry):
            bv, bg = carry
            base = g * LANES
            ps = ps_s_v[pl.ds(base, LANES)]
            pe = ps_e_v[pl.ds(base, LANES)]
            inter = jnp.maximum(
                jnp.minimum(te, pe) - jnp.maximum(ts, ps), 0.0)
            lp = pe - ps
            union = lt + lp - inter
            iou = inter / jnp.maximum(union, 1e-8)
            enclose = jnp.maximum(te, pe) - jnp.minimum(ts, ps)
            score = iou - (enclose - union) / jnp.maximum(enclose, 1e-8)
            upd = score > bv
            bv = jnp.where(upd, score, bv)
            bg = jnp.where(upd, g, bg)
            return bv, bg

        bv, bg = lax.fori_loop(
            0, N_GROUPS, g_step,
            (jnp.full((LANES,), NEG_INF, jnp.float32),
             jnp.zeros((LANES,), jnp.int32)), unroll=8)
        bi = bg * LANES + lane_iota

        bvf = jnp.maximum(bv, lax.rev(bv, (0,)))
        m = bvf[0]
        for l in range(1, LANES // 2):
            m = jnp.maximum(m, bvf[l])
        cand = jnp.where(bv == m, bi, BIG)
        candf = jnp.minimum(cand, lax.rev(cand, (0,)))
        lidx = candf[0]
        for l in range(1, LANES // 2):
            lidx = jnp.minimum(lidx, candf[l])

        parity = t % 2
        stage_v[pl.ds(0, LANES)] = jnp.full((LANES,), m, jnp.float32)
        stage_v[pl.ds(LANES, LANES)] = jnp.full(
            (LANES,), (lidx + base_n).astype(jnp.float32), jnp.float32)
        slot = parity * (16 * 2 * LANES) + s * (2 * LANES)
        pltpu.sync_copy(stage_v, sh_win.at[pl.ds(slot, 2 * LANES)])
        plsc.subcore_barrier()

        roff = parity * (16 * 2 * LANES) + bloc * (4 * 2 * LANES)
        pltpu.sync_copy(sh_win.at[pl.ds(roff, 4 * 2 * LANES)], mrg_v)
        vs = [mrg_v[pl.ds(k * 2 * LANES, LANES)] for k in range(4)]
        ifs = [mrg_v[pl.ds(k * 2 * LANES + LANES, LANES)] for k in range(4)]
        vmx = jnp.maximum(jnp.maximum(vs[0], vs[1]),
                          jnp.maximum(vs[2], vs[3]))
        bigf = jnp.float32(float(BIG))
        cands = [jnp.where(vs[k] == vmx, ifs[k], bigf) for k in range(4)]
        imin = jnp.minimum(jnp.minimum(cands[0], cands[1]),
                           jnp.minimum(cands[2], cands[3]))
        mg = vmx[0]
        gidx = imin[0].astype(jnp.int32)

        loc = gidx - base_n

        @pl.when(jnp.logical_and(loc >= 0, loc < CHUNK))
        def _():
            lane0 = lane_iota == 0
            vu = ps_e_v[pl.ds(loc, LANES)]
            ps_e_v[pl.ds(loc, LANES)] = jnp.where(lane0, minus_inf, vu)

        @pl.when(chunk == 0)
        def _():
            lane0 = lane_iota == 0
            vi = oidx_v[pl.ds(t, LANES)]
            oidx_v[pl.ds(t, LANES)] = jnp.where(lane0, gidx, vi)
            vv = oval_v[pl.ds(t, LANES)]
            oval_v[pl.ds(t, LANES)] = jnp.where(lane0, mg, vv)

        return 0

    lax.fori_loop(0, T, t_step, 0)

    @pl.when(chunk == 0)
    def _():
        pltpu.sync_copy(oidx_v, out_idx_hbm.at[b])
        pltpu.sync_copy(oval_v, out_val_hbm.at[b])


@jax.jit
def kernel(pred_logits, pred_segments, tgt_segments, prediction_duration):
    del pred_logits
    scale = prediction_duration[:, None, None]
    ps = pred_segments * scale
    ts = tgt_segments * scale
    ps_s = jnp.pad(ps[..., 0], ((0, 0), (0, N_PAD - N))).reshape(B * 4, CHUNK)
    ps_e = jnp.pad(ps[..., 1], ((0, 0), (0, N_PAD - N))).reshape(B * 4, CHUNK)
    ts_s = jnp.pad(ts[..., 0], ((0, 0), (0, T_PAD - T)))
    ts_e = jnp.pad(ts[..., 1], ((0, 0), (0, T_PAD - T)))

    mesh = plsc.VectorSubcoreMesh(core_axis_name="c", subcore_axis_name="s")
    run = pl.kernel(
        _sc_body,
        out_type=(jax.ShapeDtypeStruct((B, T_PAD), jnp.int32),
                  jax.ShapeDtypeStruct((B, T_PAD), jnp.float32)),
        mesh=mesh,
        scratch_types=[
            pltpu.VMEM((CHUNK,), jnp.float32),
            pltpu.VMEM((CHUNK + LANES,), jnp.float32),
            pltpu.VMEM((T_PAD,), jnp.float32),
            pltpu.VMEM((T_PAD,), jnp.float32),
            pltpu.VMEM((T_PAD,), jnp.int32),
            pltpu.VMEM((T_PAD,), jnp.float32),
            pltpu.VMEM((2 * LANES,), jnp.float32),
            pltpu.VMEM((4 * 2 * LANES,), jnp.float32),
            pltpu.VMEM_SHARED((2 * 16 * 2 * LANES,), jnp.float32),
            pltpu.SemaphoreType.DMA,
        ],
    )
    out_idx, out_val = run(ps_s, ps_e, ts_s, ts_e)
    return (out_idx[:, :T].astype(jnp.int64),
            out_val[:, :T])


# speedup vs baseline: 1.0567x; 1.0403x over previous
import jax
import jax.numpy as jnp
from jax import lax
from jax.experimental import pallas as pl
from jax.experimental.pallas import tpu as pltpu
from jax.experimental.pallas import tpu_sc as plsc

B, N, T = 8, 5000, 50
LANES = 16
N_PAD = 5120
CHUNK = N_PAD // 4
T_PAD = 80
N_GROUPS = CHUNK // LANES
NEG_INF = float("-inf")
BIG = 2**30
SLOT = 2 * LANES
PAR = 16 * SLOT


def _sc_body(ps_s_hbm, ps_e_hbm, ts_s_hbm, ts_e_hbm,
             out_idx_hbm, out_val_hbm,
             ps_s_v, ps_e_v, ts_s_v, ts_e_v, oidx_v, oval_v,
             stage_v, mrg_v, sh_win, sem, sem_pub, sem_rd):
    c = lax.axis_index("c")
    s = lax.axis_index("s")
    bloc = s // 4
    chunk = s % 4
    b = c * 4 + bloc
    w = b * 4 + chunk
    base_n = chunk * CHUNK

    cp1 = pltpu.make_async_copy(ps_s_hbm.at[w], ps_s_v, sem)
    cp1.start()
    cp2 = pltpu.make_async_copy(ps_e_hbm.at[w], ps_e_v.at[pl.ds(0, CHUNK)],
                                sem)
    cp2.start()
    cp3 = pltpu.make_async_copy(ts_s_hbm.at[b], ts_s_v, sem)
    cp3.start()
    cp4 = pltpu.make_async_copy(ts_e_hbm.at[b], ts_e_v, sem)
    cp4.start()

    lane_iota = lax.broadcasted_iota(jnp.int32, (LANES,), 0)
    minus_inf = jnp.float32(NEG_INF)

    stage_v[pl.ds(0, LANES)] = jnp.zeros((LANES,), jnp.float32)
    stage_v[pl.ds(LANES, LANES)] = jnp.zeros((LANES,), jnp.float32)
    pltpu.sync_copy(stage_v, sh_win.at[pl.ds(s * SLOT, SLOT)])
    pltpu.sync_copy(stage_v, sh_win.at[pl.ds(PAR + s * SLOT, SLOT)])

    cp1.wait()
    cp2.wait()
    cp3.wait()
    cp4.wait()

    def init_step(g, _):
        idxv = lane_iota + (base_n + g * LANES)
        pe = ps_e_v[pl.ds(g * LANES, LANES)]
        ps_e_v[pl.ds(g * LANES, LANES)] = jnp.where(
            idxv < N, pe, minus_inf)
        return 0

    lax.fori_loop(0, N_GROUPS, init_step, 0, unroll=8)

    def scan_row(t):
        ts = ts_s_v[pl.ds(t, LANES)][0]
        te = ts_e_v[pl.ds(t, LANES)][0]
        lt = te - ts

        def g_step(g, carry):
            bv, bg = carry
            base = g * LANES
            ps = ps_s_v[pl.ds(base, LANES)]
            pe = ps_e_v[pl.ds(base, LANES)]
            inter = jnp.maximum(
                jnp.minimum(te, pe) - jnp.maximum(ts, ps), 0.0)
            lp = pe - ps
            union = lt + lp - inter
            iou = inter / jnp.maximum(union, 1e-8)
            enclose = jnp.maximum(te, pe) - jnp.minimum(ts, ps)
            score = iou - (enclose - union) / jnp.maximum(enclose, 1e-8)
            upd = score > bv
            bv = jnp.where(upd, score, bv)
            bg = jnp.where(upd, g, bg)
            return bv, bg

        bv, bg = lax.fori_loop(
            0, N_GROUPS, g_step,
            (jnp.full((LANES,), NEG_INF, jnp.float32),
             jnp.zeros((LANES,), jnp.int32)), unroll=8)
        bi = bg * LANES + lane_iota

        bvf = jnp.maximum(bv, lax.rev(bv, (0,)))
        m = bvf[0]
        for l in range(1, LANES // 2):
            m = jnp.maximum(m, bvf[l])
        cand = jnp.where(bv == m, bi, BIG)
        candf = jnp.minimum(cand, lax.rev(cand, (0,)))
        lidx = candf[0]
        for l in range(1, LANES // 2):
            lidx = jnp.minimum(lidx, candf[l])
        return m, lidx + base_n

    def publish_stage(m, gidx):
        stage_v[pl.ds(0, LANES)] = jnp.full((LANES,), m, jnp.float32)
        stage_v[pl.ds(LANES, LANES)] = jnp.full(
            (LANES,), gidx.astype(jnp.float32), jnp.float32)

    def t_step(t, _):
        m, gidx_cand = scan_row(t)

        parity = t % 2
        parity_prev = 1 - parity

        @pl.when(t > 0)
        def _():
            pltpu.make_async_copy(
                stage_v,
                sh_win.at[pl.ds(parity_prev * PAR + s * SLOT, SLOT)],
                sem_pub).wait()

        plsc.subcore_barrier()

        roff = parity_prev * PAR + bloc * (4 * SLOT)
        rd = pltpu.make_async_copy(
            sh_win.at[pl.ds(roff, 4 * SLOT)], mrg_v, sem_rd)
        rd.start()
        rd.wait()

        vs = [mrg_v[pl.ds(k * SLOT, LANES)] for k in range(4)]
        ifs = [mrg_v[pl.ds(k * SLOT + LANES, LANES)] for k in range(4)]
        vmx = jnp.maximum(jnp.maximum(vs[0], vs[1]),
                          jnp.maximum(vs[2], vs[3]))
        bigf = jnp.float32(float(BIG))
        cands = [jnp.where(vs[k] == vmx, ifs[k], bigf) for k in range(4)]
        imin = jnp.minimum(jnp.minimum(cands[0], cands[1]),
                           jnp.minimum(cands[2], cands[3]))
        val_prev = vmx[0]
        w_raw = imin[0].astype(jnp.int32)
        w_prev = jnp.where(t > 0, w_raw, jnp.int32(-1))

        @pl.when(jnp.logical_and(t > 0, chunk == 0))
        def _():
            lane0 = lane_iota == 0
            vi = oidx_v[pl.ds(t - 1, LANES)]
            oidx_v[pl.ds(t - 1, LANES)] = jnp.where(lane0, w_prev, vi)
            vv = oval_v[pl.ds(t - 1, LANES)]
            oval_v[pl.ds(t - 1, LANES)] = jnp.where(lane0, val_prev, vv)

        loc = w_prev - base_n
        own = jnp.logical_and(loc >= 0, loc < CHUNK)

        @pl.when(own)
        def _():
            lane0 = lane_iota == 0
            vu = ps_e_v[pl.ds(loc, LANES)]
            ps_e_v[pl.ds(loc, LANES)] = jnp.where(lane0, minus_inf, vu)

        publish_stage(m, gidx_cand)

        @pl.when(gidx_cand == w_prev)
        def _():
            m2, gidx2 = scan_row(t)
            publish_stage(m2, gidx2)

        pltpu.make_async_copy(
            stage_v, sh_win.at[pl.ds(parity * PAR + s * SLOT, SLOT)],
            sem_pub).start()
        return 0

    lax.fori_loop(0, T, t_step, 0)

    parity_last = (T - 1) % 2
    pltpu.make_async_copy(
        stage_v, sh_win.at[pl.ds(parity_last * PAR + s * SLOT, SLOT)],
        sem_pub).wait()
    plsc.subcore_barrier()

    @pl.when(chunk == 0)
    def _():
        roff = parity_last * PAR + bloc * (4 * SLOT)
        pltpu.sync_copy(sh_win.at[pl.ds(roff, 4 * SLOT)], mrg_v)
        vs = [mrg_v[pl.ds(k * SLOT, LANES)] for k in range(4)]
        ifs = [mrg_v[pl.ds(k * SLOT + LANES, LANES)] for k in range(4)]
        vmx = jnp.maximum(jnp.maximum(vs[0], vs[1]),
                          jnp.maximum(vs[2], vs[3]))
        bigf = jnp.float32(float(BIG))
        cands = [jnp.where(vs[k] == vmx, ifs[k], bigf) for k in range(4)]
        imin = jnp.minimum(jnp.minimum(cands[0], cands[1]),
                           jnp.minimum(cands[2], cands[3]))
        lane0 = lane_iota == 0
        vi = oidx_v[pl.ds(T - 1, LANES)]
        oidx_v[pl.ds(T - 1, LANES)] = jnp.where(
            lane0, imin[0].astype(jnp.int32), vi)
        vv = oval_v[pl.ds(T - 1, LANES)]
        oval_v[pl.ds(T - 1, LANES)] = jnp.where(lane0, vmx[0], vv)
        pltpu.sync_copy(oidx_v, out_idx_hbm.at[b])
        pltpu.sync_copy(oval_v, out_val_hbm.at[b])


@jax.jit
def kernel(pred_logits, pred_segments, tgt_segments, prediction_duration):
    del pred_logits
    scale = prediction_duration[:, None, None]
    ps = pred_segments * scale
    ts = tgt_segments * scale
    ps_s = jnp.pad(ps[..., 0], ((0, 0), (0, N_PAD - N))).reshape(B * 4, CHUNK)
    ps_e = jnp.pad(ps[..., 1], ((0, 0), (0, N_PAD - N))).reshape(B * 4, CHUNK)
    ts_s = jnp.pad(ts[..., 0], ((0, 0), (0, T_PAD - T)))
    ts_e = jnp.pad(ts[..., 1], ((0, 0), (0, T_PAD - T)))

    mesh = plsc.VectorSubcoreMesh(core_axis_name="c", subcore_axis_name="s")
    run = pl.kernel(
        _sc_body,
        out_type=(jax.ShapeDtypeStruct((B, T_PAD), jnp.int32),
                  jax.ShapeDtypeStruct((B, T_PAD), jnp.float32)),
        mesh=mesh,
        scratch_types=[
            pltpu.VMEM((CHUNK,), jnp.float32),
            pltpu.VMEM((CHUNK + LANES,), jnp.float32),
            pltpu.VMEM((T_PAD,), jnp.float32),
            pltpu.VMEM((T_PAD,), jnp.float32),
            pltpu.VMEM((T_PAD,), jnp.int32),
            pltpu.VMEM((T_PAD,), jnp.float32),
            pltpu.VMEM((SLOT,), jnp.float32),
            pltpu.VMEM((4 * SLOT,), jnp.float32),
            pltpu.VMEM_SHARED((2 * PAR,), jnp.float32),
            pltpu.SemaphoreType.DMA,
            pltpu.SemaphoreType.DMA,
            pltpu.SemaphoreType.DMA,
        ],
    )
    out_idx, out_val = run(ps_s, ps_e, ts_s, ts_e)
    return (out_idx[:, :T].astype(jnp.int64),
            out_val[:, :T])


# speedup vs baseline: 1.0675x; 1.0102x over previous
import jax
import jax.numpy as jnp
from jax import lax
from jax.experimental import pallas as pl
from jax.experimental.pallas import tpu as pltpu
from jax.experimental.pallas import tpu_sc as plsc

B, N, T = 8, 5000, 50
LANES = 16
N_PAD = 5120
CHUNK = N_PAD // 4
T_PAD = 80
N_GROUPS = CHUNK // LANES
NEG_INF = float("-inf")
BIG = 2**30
SLOT = 2 * LANES
PAR = 16 * SLOT


def _sc_body(ps_s_hbm, ps_e_hbm, ts_s_hbm, ts_e_hbm,
             out_idx_hbm, out_val_hbm,
             ps_s_v, ps_e_v, ts_s_v, ts_e_v, oidx_v, oval_v,
             stage_v, mrg_v, sh_win, sem, sem_pub, sem_rd):
    c = lax.axis_index("c")
    s = lax.axis_index("s")
    bloc = s // 4
    chunk = s % 4
    b = c * 4 + bloc
    w = b * 4 + chunk
    base_n = chunk * CHUNK

    cp1 = pltpu.make_async_copy(ps_s_hbm.at[w], ps_s_v, sem)
    cp1.start()
    cp2 = pltpu.make_async_copy(ps_e_hbm.at[w], ps_e_v.at[pl.ds(0, CHUNK)],
                                sem)
    cp2.start()
    cp3 = pltpu.make_async_copy(ts_s_hbm.at[b], ts_s_v, sem)
    cp3.start()
    cp4 = pltpu.make_async_copy(ts_e_hbm.at[b], ts_e_v, sem)
    cp4.start()

    lane_iota = lax.broadcasted_iota(jnp.int32, (LANES,), 0)
    minus_inf = jnp.float32(NEG_INF)

    stage_v[pl.ds(0, LANES)] = jnp.zeros((LANES,), jnp.float32)
    stage_v[pl.ds(LANES, LANES)] = jnp.zeros((LANES,), jnp.float32)
    pltpu.sync_copy(stage_v, sh_win.at[pl.ds(s * SLOT, SLOT)])
    pltpu.sync_copy(stage_v, sh_win.at[pl.ds(PAR + s * SLOT, SLOT)])

    cp1.wait()
    cp2.wait()
    cp3.wait()
    cp4.wait()

    def init_step(g, _):
        idxv = lane_iota + (base_n + g * LANES)
        pe = ps_e_v[pl.ds(g * LANES, LANES)]
        ps_e_v[pl.ds(g * LANES, LANES)] = jnp.where(
            idxv < N, pe, minus_inf)
        return 0

    lax.fori_loop(0, N_GROUPS, init_step, 0, unroll=8)

    def scan_groups(t):
        ts = ts_s_v[pl.ds(t, LANES)][0]
        te = ts_e_v[pl.ds(t, LANES)][0]
        lt = te - ts

        def g_step(g, carry):
            bv, bg = carry
            base = g * LANES
            ps = ps_s_v[pl.ds(base, LANES)]
            pe = ps_e_v[pl.ds(base, LANES)]
            inter = jnp.maximum(
                jnp.minimum(te, pe) - jnp.maximum(ts, ps), 0.0)
            lp = pe - ps
            union = lt + lp - inter
            iou = inter / jnp.maximum(union, 1e-8)
            enclose = jnp.maximum(te, pe) - jnp.minimum(ts, ps)
            score = iou - (enclose - union) / jnp.maximum(enclose, 1e-8)
            upd = score > bv
            bv = jnp.where(upd, score, bv)
            bg = jnp.where(upd, g, bg)
            return bv, bg

        bv, bg = lax.fori_loop(
            0, N_GROUPS, g_step,
            (jnp.full((LANES,), NEG_INF, jnp.float32),
             jnp.zeros((LANES,), jnp.int32)), unroll=8)
        return bv, bg

    def lane_reduce(bv, bg):
        bi = bg * LANES + lane_iota

        bvf = jnp.maximum(bv, lax.rev(bv, (0,)))
        m = bvf[0]
        for l in range(1, LANES // 2):
            m = jnp.maximum(m, bvf[l])
        cand = jnp.where(bv == m, bi, BIG)
        candf = jnp.minimum(cand, lax.rev(cand, (0,)))
        lidx = candf[0]
        for l in range(1, LANES // 2):
            lidx = jnp.minimum(lidx, candf[l])
        return m, lidx + base_n

    def publish_stage(m, gidx):
        stage_v[pl.ds(0, LANES)] = jnp.full((LANES,), m, jnp.float32)
        stage_v[pl.ds(LANES, LANES)] = jnp.full(
            (LANES,), gidx.astype(jnp.float32), jnp.float32)

    def t_step(t, _):
        bv, bg = scan_groups(t)

        parity = t % 2
        parity_prev = 1 - parity

        @pl.when(t > 0)
        def _():
            pltpu.make_async_copy(
                stage_v,
                sh_win.at[pl.ds(parity_prev * PAR + s * SLOT, SLOT)],
                sem_pub).wait()

        plsc.subcore_barrier()

        roff = parity_prev * PAR + bloc * (4 * SLOT)
        rd = pltpu.make_async_copy(
            sh_win.at[pl.ds(roff, 4 * SLOT)], mrg_v, sem_rd)
        rd.start()
        m, gidx_cand = lane_reduce(bv, bg)
        rd.wait()

        vs = [mrg_v[pl.ds(k * SLOT, LANES)] for k in range(4)]
        ifs = [mrg_v[pl.ds(k * SLOT + LANES, LANES)] for k in range(4)]
        vmx = jnp.maximum(jnp.maximum(vs[0], vs[1]),
                          jnp.maximum(vs[2], vs[3]))
        bigf = jnp.float32(float(BIG))
        cands = [jnp.where(vs[k] == vmx, ifs[k], bigf) for k in range(4)]
        imin = jnp.minimum(jnp.minimum(cands[0], cands[1]),
                           jnp.minimum(cands[2], cands[3]))
        val_prev = vmx[0]
        w_raw = imin[0].astype(jnp.int32)
        w_prev = jnp.where(t > 0, w_raw, jnp.int32(-1))

        @pl.when(jnp.logical_and(t > 0, chunk == 0))
        def _():
            lane0 = lane_iota == 0
            vi = oidx_v[pl.ds(t - 1, LANES)]
            oidx_v[pl.ds(t - 1, LANES)] = jnp.where(lane0, w_prev, vi)
            vv = oval_v[pl.ds(t - 1, LANES)]
            oval_v[pl.ds(t - 1, LANES)] = jnp.where(lane0, val_prev, vv)

        loc = w_prev - base_n
        own = jnp.logical_and(loc >= 0, loc < CHUNK)

        @pl.when(own)
        def _():
            lane0 = lane_iota == 0
            vu = ps_e_v[pl.ds(loc, LANES)]
            ps_e_v[pl.ds(loc, LANES)] = jnp.where(lane0, minus_inf, vu)

        publish_stage(m, gidx_cand)

        @pl.when(gidx_cand == w_prev)
        def _():
            m2, gidx2 = lane_reduce(*scan_groups(t))
            publish_stage(m2, gidx2)

        pltpu.make_async_copy(
            stage_v, sh_win.at[pl.ds(parity * PAR + s * SLOT, SLOT)],
            sem_pub).start()
        return 0

    lax.fori_loop(0, T, t_step, 0)

    parity_last = (T - 1) % 2
    pltpu.make_async_copy(
        stage_v, sh_win.at[pl.ds(parity_last * PAR + s * SLOT, SLOT)],
        sem_pub).wait()
    plsc.subcore_barrier()

    @pl.when(chunk == 0)
    def _():
        roff = parity_last * PAR + bloc * (4 * SLOT)
        pltpu.sync_copy(sh_win.at[pl.ds(roff, 4 * SLOT)], mrg_v)
        vs = [mrg_v[pl.ds(k * SLOT, LANES)] for k in range(4)]
        ifs = [mrg_v[pl.ds(k * SLOT + LANES, LANES)] for k in range(4)]
        vmx = jnp.maximum(jnp.maximum(vs[0], vs[1]),
                          jnp.maximum(vs[2], vs[3]))
        bigf = jnp.float32(float(BIG))
        cands = [jnp.where(vs[k] == vmx, ifs[k], bigf) for k in range(4)]
        imin = jnp.minimum(jnp.minimum(cands[0], cands[1]),
                           jnp.minimum(cands[2], cands[3]))
        lane0 = lane_iota == 0
        vi = oidx_v[pl.ds(T - 1, LANES)]
        oidx_v[pl.ds(T - 1, LANES)] = jnp.where(
            lane0, imin[0].astype(jnp.int32), vi)
        vv = oval_v[pl.ds(T - 1, LANES)]
        oval_v[pl.ds(T - 1, LANES)] = jnp.where(lane0, vmx[0], vv)
        pltpu.sync_copy(oidx_v, out_idx_hbm.at[b])
        pltpu.sync_copy(oval_v, out_val_hbm.at[b])


@jax.jit
def kernel(pred_logits, pred_segments, tgt_segments, prediction_duration):
    del pred_logits
    scale = prediction_duration[:, None, None]
    ps = pred_segments * scale
    ts = tgt_segments * scale
    ps_s = jnp.pad(ps[..., 0], ((0, 0), (0, N_PAD - N))).reshape(B * 4, CHUNK)
    ps_e = jnp.pad(ps[..., 1], ((0, 0), (0, N_PAD - N))).reshape(B * 4, CHUNK)
    ts_s = jnp.pad(ts[..., 0], ((0, 0), (0, T_PAD - T)))
    ts_e = jnp.pad(ts[..., 1], ((0, 0), (0, T_PAD - T)))

    mesh = plsc.VectorSubcoreMesh(core_axis_name="c", subcore_axis_name="s")
    run = pl.kernel(
        _sc_body,
        out_type=(jax.ShapeDtypeStruct((B, T_PAD), jnp.int32),
                  jax.ShapeDtypeStruct((B, T_PAD), jnp.float32)),
        mesh=mesh,
        scratch_types=[
            pltpu.VMEM((CHUNK,), jnp.float32),
            pltpu.VMEM((CHUNK + LANES,), jnp.float32),
            pltpu.VMEM((T_PAD,), jnp.float32),
            pltpu.VMEM((T_PAD,), jnp.float32),
            pltpu.VMEM((T_PAD,), jnp.int32),
            pltpu.VMEM((T_PAD,), jnp.float32),
            pltpu.VMEM((SLOT,), jnp.float32),
            pltpu.VMEM((4 * SLOT,), jnp.float32),
            pltpu.VMEM_SHARED((2 * PAR,), jnp.float32),
            pltpu.SemaphoreType.DMA,
            pltpu.SemaphoreType.DMA,
            pltpu.SemaphoreType.DMA,
        ],
    )
    out_idx, out_val = run(ps_s, ps_e, ts_s, ts_e)
    return (out_idx[:, :T].astype(jnp.int64),
            out_val[:, :T])


# speedup vs baseline: 1.0761x; 1.0081x over previous
import jax
import jax.numpy as jnp
from jax import lax
from jax.experimental import pallas as pl
from jax.experimental.pallas import tpu as pltpu
from jax.experimental.pallas import tpu_sc as plsc

B, N, T = 8, 5000, 50
LANES = 16
N_PAD = 5120
CHUNK = N_PAD // 4
T_PAD = 80
N_GROUPS = CHUNK // LANES
NEG_INF = float("-inf")
BIG = 2**30
SLOT = 2 * LANES
PAR = 16 * SLOT


def _sc_body(ps_s_hbm, ps_e_hbm, ts_s_hbm, ts_e_hbm,
             out_idx_hbm, out_val_hbm,
             ps_s_v, ps_e_v, lp_v, ts_s_v, ts_e_v, oidx_v, oval_v,
             stage_v, mrg_v, sh_win, sem, sem_pub, sem_rd):
    c = lax.axis_index("c")
    s = lax.axis_index("s")
    bloc = s // 4
    chunk = s % 4
    b = c * 4 + bloc
    w = b * 4 + chunk
    base_n = chunk * CHUNK

    cp1 = pltpu.make_async_copy(ps_s_hbm.at[w], ps_s_v, sem)
    cp1.start()
    cp2 = pltpu.make_async_copy(ps_e_hbm.at[w], ps_e_v.at[pl.ds(0, CHUNK)],
                                sem)
    cp2.start()
    cp3 = pltpu.make_async_copy(ts_s_hbm.at[b], ts_s_v, sem)
    cp3.start()
    cp4 = pltpu.make_async_copy(ts_e_hbm.at[b], ts_e_v, sem)
    cp4.start()

    lane_iota = lax.broadcasted_iota(jnp.int32, (LANES,), 0)
    minus_inf = jnp.float32(NEG_INF)

    stage_v[pl.ds(0, LANES)] = jnp.zeros((LANES,), jnp.float32)
    stage_v[pl.ds(LANES, LANES)] = jnp.zeros((LANES,), jnp.float32)
    pltpu.sync_copy(stage_v, sh_win.at[pl.ds(s * SLOT, SLOT)])
    pltpu.sync_copy(stage_v, sh_win.at[pl.ds(PAR + s * SLOT, SLOT)])

    cp1.wait()
    cp2.wait()
    cp3.wait()
    cp4.wait()

    def init_step(g, _):
        idxv = lane_iota + (base_n + g * LANES)
        ps = ps_s_v[pl.ds(g * LANES, LANES)]
        pe = ps_e_v[pl.ds(g * LANES, LANES)]
        valid = idxv < N
        lp_v[pl.ds(g * LANES, LANES)] = jnp.where(valid, pe - ps, minus_inf)
        ps_e_v[pl.ds(g * LANES, LANES)] = jnp.where(valid, pe, minus_inf)
        return 0

    lax.fori_loop(0, N_GROUPS, init_step, 0, unroll=8)

    def scan_groups(t):
        ts = ts_s_v[pl.ds(t, LANES)][0]
        te = ts_e_v[pl.ds(t, LANES)][0]
        lt = te - ts

        def g_step(g, carry):
            bv, bg = carry
            base = g * LANES
            ps = ps_s_v[pl.ds(base, LANES)]
            pe = ps_e_v[pl.ds(base, LANES)]
            inter = jnp.maximum(
                jnp.minimum(te, pe) - jnp.maximum(ts, ps), 0.0)
            lp = lp_v[pl.ds(base, LANES)]
            union = lt + lp - inter
            iou = inter / jnp.maximum(union, 1e-8)
            enclose = jnp.maximum(te, pe) - jnp.minimum(ts, ps)
            score = iou - (enclose - union) / jnp.maximum(enclose, 1e-8)
            upd = score > bv
            bv = jnp.where(upd, score, bv)
            bg = jnp.where(upd, g, bg)
            return bv, bg

        bv, bg = lax.fori_loop(
            0, N_GROUPS, g_step,
            (jnp.full((LANES,), NEG_INF, jnp.float32),
             jnp.zeros((LANES,), jnp.int32)), unroll=8)
        return bv, bg

    def lane_reduce(bv, bg):
        bi = bg * LANES + lane_iota

        bvf = jnp.maximum(bv, lax.rev(bv, (0,)))
        m = bvf[0]
        for l in range(1, LANES // 2):
            m = jnp.maximum(m, bvf[l])
        cand = jnp.where(bv == m, bi, BIG)
        candf = jnp.minimum(cand, lax.rev(cand, (0,)))
        lidx = candf[0]
        for l in range(1, LANES // 2):
            lidx = jnp.minimum(lidx, candf[l])
        return m, lidx + base_n

    def publish_stage(m, gidx):
        stage_v[pl.ds(0, LANES)] = jnp.full((LANES,), m, jnp.float32)
        stage_v[pl.ds(LANES, LANES)] = jnp.full(
            (LANES,), gidx.astype(jnp.float32), jnp.float32)

    def t_step(t, _):
        bv, bg = scan_groups(t)

        parity = t % 2
        parity_prev = 1 - parity

        @pl.when(t > 0)
        def _():
            pltpu.make_async_copy(
                stage_v,
                sh_win.at[pl.ds(parity_prev * PAR + s * SLOT, SLOT)],
                sem_pub).wait()

        plsc.subcore_barrier()

        roff = parity_prev * PAR + bloc * (4 * SLOT)
        rd = pltpu.make_async_copy(
            sh_win.at[pl.ds(roff, 4 * SLOT)], mrg_v, sem_rd)
        rd.start()
        m, gidx_cand = lane_reduce(bv, bg)
        rd.wait()

        vs = [mrg_v[pl.ds(k * SLOT, LANES)] for k in range(4)]
        ifs = [mrg_v[pl.ds(k * SLOT + LANES, LANES)] for k in range(4)]
        vmx = jnp.maximum(jnp.maximum(vs[0], vs[1]),
                          jnp.maximum(vs[2], vs[3]))
        bigf = jnp.float32(float(BIG))
        cands = [jnp.where(vs[k] == vmx, ifs[k], bigf) for k in range(4)]
        imin = jnp.minimum(jnp.minimum(cands[0], cands[1]),
                           jnp.minimum(cands[2], cands[3]))
        val_prev = vmx[0]
        w_raw = imin[0].astype(jnp.int32)
        w_prev = jnp.where(t > 0, w_raw, jnp.int32(-1))

        @pl.when(jnp.logical_and(t > 0, chunk == 0))
        def _():
            lane0 = lane_iota == 0
            vi = oidx_v[pl.ds(t - 1, LANES)]
            oidx_v[pl.ds(t - 1, LANES)] = jnp.where(lane0, w_prev, vi)
            vv = oval_v[pl.ds(t - 1, LANES)]
            oval_v[pl.ds(t - 1, LANES)] = jnp.where(lane0, val_prev, vv)

        loc = w_prev - base_n
        own = jnp.logical_and(loc >= 0, loc < CHUNK)

        @pl.when(own)
        def _():
            lane0 = lane_iota == 0
            vu = ps_e_v[pl.ds(loc, LANES)]
            ps_e_v[pl.ds(loc, LANES)] = jnp.where(lane0, minus_inf, vu)
            vl = lp_v[pl.ds(loc, LANES)]
            lp_v[pl.ds(loc, LANES)] = jnp.where(lane0, minus_inf, vl)

        publish_stage(m, gidx_cand)

        @pl.when(gidx_cand == w_prev)
        def _():
            m2, gidx2 = lane_reduce(*scan_groups(t))
            publish_stage(m2, gidx2)

        pltpu.make_async_copy(
            stage_v, sh_win.at[pl.ds(parity * PAR + s * SLOT, SLOT)],
            sem_pub).start()
        return 0

    lax.fori_loop(0, T, t_step, 0)

    parity_last = (T - 1) % 2
    pltpu.make_async_copy(
        stage_v, sh_win.at[pl.ds(parity_last * PAR + s * SLOT, SLOT)],
        sem_pub).wait()
    plsc.subcore_barrier()

    @pl.when(chunk == 0)
    def _():
        roff = parity_last * PAR + bloc * (4 * SLOT)
        pltpu.sync_copy(sh_win.at[pl.ds(roff, 4 * SLOT)], mrg_v)
        vs = [mrg_v[pl.ds(k * SLOT, LANES)] for k in range(4)]
        ifs = [mrg_v[pl.ds(k * SLOT + LANES, LANES)] for k in range(4)]
        vmx = jnp.maximum(jnp.maximum(vs[0], vs[1]),
                          jnp.maximum(vs[2], vs[3]))
        bigf = jnp.float32(float(BIG))
        cands = [jnp.where(vs[k] == vmx, ifs[k], bigf) for k in range(4)]
        imin = jnp.minimum(jnp.minimum(cands[0], cands[1]),
                           jnp.minimum(cands[2], cands[3]))
        lane0 = lane_iota == 0
        vi = oidx_v[pl.ds(T - 1, LANES)]
        oidx_v[pl.ds(T - 1, LANES)] = jnp.where(
            lane0, imin[0].astype(jnp.int32), vi)
        vv = oval_v[pl.ds(T - 1, LANES)]
        oval_v[pl.ds(T - 1, LANES)] = jnp.where(lane0, vmx[0], vv)
        pltpu.sync_copy(oidx_v, out_idx_hbm.at[b])
        pltpu.sync_copy(oval_v, out_val_hbm.at[b])


@jax.jit
def kernel(pred_logits, pred_segments, tgt_segments, prediction_duration):
    del pred_logits
    scale = prediction_duration[:, None, None]
    ps = pred_segments * scale
    ts = tgt_segments * scale
    ps_s = jnp.pad(ps[..., 0], ((0, 0), (0, N_PAD - N))).reshape(B * 4, CHUNK)
    ps_e = jnp.pad(ps[..., 1], ((0, 0), (0, N_PAD - N))).reshape(B * 4, CHUNK)
    ts_s = jnp.pad(ts[..., 0], ((0, 0), (0, T_PAD - T)))
    ts_e = jnp.pad(ts[..., 1], ((0, 0), (0, T_PAD - T)))

    mesh = plsc.VectorSubcoreMesh(core_axis_name="c", subcore_axis_name="s")
    run = pl.kernel(
        _sc_body,
        out_type=(jax.ShapeDtypeStruct((B, T_PAD), jnp.int32),
                  jax.ShapeDtypeStruct((B, T_PAD), jnp.float32)),
        mesh=mesh,
        scratch_types=[
            pltpu.VMEM((CHUNK,), jnp.float32),
            pltpu.VMEM((CHUNK + LANES,), jnp.float32),
            pltpu.VMEM((CHUNK + LANES,), jnp.float32),
            pltpu.VMEM((T_PAD,), jnp.float32),
            pltpu.VMEM((T_PAD,), jnp.float32),
            pltpu.VMEM((T_PAD,), jnp.int32),
            pltpu.VMEM((T_PAD,), jnp.float32),
            pltpu.VMEM((SLOT,), jnp.float32),
            pltpu.VMEM((4 * SLOT,), jnp.float32),
            pltpu.VMEM_SHARED((2 * PAR,), jnp.float32),
            pltpu.SemaphoreType.DMA,
            pltpu.SemaphoreType.DMA,
            pltpu.SemaphoreType.DMA,
        ],
    )
    out_idx, out_val = run(ps_s, ps_e, ts_s, ts_e)
    return (out_idx[:, :T].astype(jnp.int64),
            out_val[:, :T])
